# baseline (speedup 1.0000x reference)
import functools

import jax
import jax.numpy as jnp
from jax.experimental import pallas as pl
from jax.experimental.pallas import tpu as pltpu

_HALF_M = 1024
_SUPER_M = 2 * _HALF_M
_DEPTH = 2


def _router_body(xa_ref, x_hbm, w_ref, o_ref, bbuf, bsem):
    sup = pl.program_id(0)
    n_super = pl.num_programs(0)
    dn = (((1,), (1,)), ((), ()))

    def bcopy(sup_idx, slot):
        row = sup_idx * _SUPER_M + _HALF_M
        return pltpu.make_async_copy(
            x_hbm.at[pl.ds(row, _HALF_M), :],
            bbuf.at[slot],
            bsem.at[slot],
        )

    @pl.when(sup == 0)
    def _():
        for d in range(_DEPTH):
            bcopy(d, d).start()

    slot = jax.lax.rem(sup, _DEPTH)
    o_ref[pl.ds(0, _HALF_M), :] = jax.lax.dot_general(
        xa_ref[...], w_ref[...], dimension_numbers=dn,
        preferred_element_type=jnp.float32)
    bcopy(sup, slot).wait()
    o_ref[pl.ds(_HALF_M, _HALF_M), :] = jax.lax.dot_general(
        bbuf[slot], w_ref[...], dimension_numbers=dn,
        preferred_element_type=jnp.float32)

    @pl.when(sup + _DEPTH < n_super)
    def _():
        bcopy(sup + _DEPTH, slot).start()


@functools.partial(jax.jit, static_argnames=())
def kernel(x, W):
    B, T, D = x.shape
    E = W.shape[0]
    M = B * T
    x2 = x.reshape(M, D)
    grid = (M // _SUPER_M,)
    out = pl.pallas_call(
        _router_body,
        grid=grid,
        in_specs=[
            pl.BlockSpec((_HALF_M, D), lambda i: (2 * i, 0)),
            pl.BlockSpec(memory_space=pltpu.MemorySpace.HBM),
            pl.BlockSpec((E, D), lambda i: (0, 0)),
        ],
        out_specs=pl.BlockSpec((_SUPER_M, E), lambda i: (i, 0)),
        out_shape=jax.ShapeDtypeStruct((M, E), jnp.float32),
        scratch_shapes=[
            pltpu.VMEM((_DEPTH, _HALF_M, D), jnp.float32),
            pltpu.SemaphoreType.DMA((_DEPTH,)),
        ],
        compiler_params=pltpu.CompilerParams(
            dimension_semantics=("arbitrary",),
        ),
    )(x2, x2, W)
    return out.reshape(B, T, E)


# speedup vs baseline: 1.0736x; 1.0736x over previous
import functools

import jax
import jax.numpy as jnp
from jax.experimental import pallas as pl
from jax.experimental.pallas import tpu as pltpu

_BLOCK_M = 1024


def _router_body(x_hbm, w_ref, o_ref):
    M = x_hbm.shape[0]
    D = x_hbm.shape[1]
    E = w_ref.shape[0]
    n = M // _BLOCK_M
    dn = (((1,), (1,)), ((), ()))

    def step(x_tile, o_tile):
        o_tile[...] = jax.lax.dot_general(
            x_tile[...], w_ref[...], dimension_numbers=dn,
            preferred_element_type=jnp.float32)

    pltpu.emit_pipeline(
        step,
        grid=(n,),
        in_specs=[pl.BlockSpec((_BLOCK_M, D), lambda i: (i, 0))],
        out_specs=[pl.BlockSpec((_BLOCK_M, E), lambda i: (i, 0))],
    )(x_hbm, o_ref)


@functools.partial(jax.jit, static_argnames=())
def kernel(x, W):
    B, T, D = x.shape
    E = W.shape[0]
    M = B * T
    x2 = x.reshape(M, D)
    out = pl.pallas_call(
        _router_body,
        in_specs=[
            pl.BlockSpec(memory_space=pltpu.MemorySpace.HBM),
            pl.BlockSpec(memory_space=pltpu.VMEM),
        ],
        out_specs=pl.BlockSpec(memory_space=pltpu.MemorySpace.HBM),
        out_shape=jax.ShapeDtypeStruct((M, E), jnp.float32),
    )(x2, W)
    return out.reshape(B, T, E)


# speedup vs baseline: 1.0949x; 1.0199x over previous
import functools

import jax
import jax.numpy as jnp
from jax.experimental import pallas as pl
from jax.experimental.pallas import tpu as pltpu

_BLOCK_M = 1024


def _router_block(x_ref, w_ref, o_ref):
    o_ref[...] = jax.lax.dot_general(
        x_ref[...],
        w_ref[...],
        dimension_numbers=(((1,), (1,)), ((), ())),
        preferred_element_type=jnp.float32,
    )


@functools.partial(jax.jit, static_argnames=())
def kernel(x, W):
    B, T, D = x.shape
    E = W.shape[0]
    M = B * T
    x2 = x.reshape(M, D)
    grid = (M // _BLOCK_M,)
    out = pl.pallas_call(
        _router_block,
        grid=grid,
        in_specs=[
            pl.BlockSpec((_BLOCK_M, D), lambda i: (i, 0)),
            pl.BlockSpec((E, D), lambda i: (0, 0)),
        ],
        out_specs=pl.BlockSpec((_BLOCK_M, E), lambda i: (i, 0)),
        out_shape=jax.ShapeDtypeStruct((M, E), jnp.float32),
        compiler_params=pltpu.CompilerParams(
            dimension_semantics=("arbitrary",),
        ),
    )(x2, W)
    return out.reshape(B, T, E)
